# baseline (speedup 1.0000x reference)
import functools

import jax
import jax.numpy as jnp
from jax import lax
from jax.experimental import pallas as pl
from jax.experimental.pallas import tpu as pltpu
from jax.experimental.pallas import tpu_sc as plsc

N = 8192
D = 1024
E = 8
H = 4096
CAP = 2048
SLOTS = E * CAP
SENT = SLOTS
SLOTS_PAD = SLOTS + 16
BT = 256
NSTEPS = N // BT
BH = 256
EPAD = 128

D2 = D // 2
NC, NS = 2, 16
NW = NC * NS
TPW = N // NW
SPW = SLOTS // NW
CH = 32


def _f32_view(a):
    s = a.shape
    return lax.bitcast_convert_type(
        a.reshape(*s[:-1], s[-1] // 2, 2), jnp.float32)


def _bf16_view(a):
    s = a.shape
    return lax.bitcast_convert_type(a, jnp.bfloat16).reshape(*s[:-1], s[-1] * 2)


def _gating_body(x_ref, wg_ref, p1_ref, e2_ref, r2_ref, g1_ref, g2_ref,
                 cnt_ref, carry):
    step = pl.program_id(0)
    xb = x_ref[...]
    wgp = wg_ref[...]
    logits = jnp.dot(xb, wgp, preferred_element_type=jnp.float32)
    col = lax.broadcasted_iota(jnp.int32, (BT, EPAD), 1)
    validc = col < E
    neg = jnp.float32(-1e30)
    lg = jnp.where(validc, logits, neg)
    a1 = jnp.argmax(lg, axis=1).astype(jnp.int32)
    lg2 = jnp.where(col == a1[:, None], neg, lg)
    a2 = jnp.argmax(lg2, axis=1).astype(jnp.int32)
    mx = jnp.max(lg, axis=1, keepdims=True)
    ex = jnp.where(validc, jnp.exp(lg - mx), 0.0)
    den = jnp.sum(ex, axis=1)
    g1 = jnp.max(ex, axis=1) / den
    g2 = jnp.max(jnp.where(col == a1[:, None], 0.0, ex), axis=1) / den
    m1 = (col == a1[:, None]).astype(jnp.float32)
    m2 = (col == a2[:, None]).astype(jnp.float32)
    row_i = lax.broadcasted_iota(jnp.int32, (BT, BT), 0)
    col_i = lax.broadcasted_iota(jnp.int32, (BT, BT), 1)
    tri = (row_i >= col_i).astype(jnp.float32)

    @pl.when(step == 0)
    def _():
        carry[...] = jnp.zeros_like(carry)

    c1 = jnp.dot(tri, m1, preferred_element_type=jnp.float32) + carry[0:1, :]
    c2 = jnp.dot(tri, m2, preferred_element_type=jnp.float32) + carry[1:2, :]
    l1 = jnp.sum((c1 - 1.0) * m1, axis=1).astype(jnp.int32)
    r2i = jnp.sum(c2 * m2, axis=1).astype(jnp.int32)
    p1 = jnp.where(l1 < CAP, a1 * CAP + l1, SENT)
    carry[0:1, :] = c1[BT - 1:BT, :]
    carry[1:2, :] = c2[BT - 1:BT, :]
    p1_ref[...] = p1.reshape(1, 1, BT)
    e2_ref[...] = a2.reshape(1, 1, BT)
    r2_ref[...] = r2i.reshape(1, 1, BT)
    g1_ref[...] = g1.reshape(1, 1, BT)
    g2_ref[...] = g2.reshape(1, 1, BT)
    cnt_ref[...] = jnp.broadcast_to(c1[BT - 1:BT, :], (8, EPAD)).astype(jnp.int32)


def _gating_call(x, wgp):
    return pl.pallas_call(
        _gating_body,
        grid=(NSTEPS,),
        in_specs=[
            pl.BlockSpec((BT, D), lambda i: (i, 0)),
            pl.BlockSpec((D, EPAD), lambda i: (0, 0)),
        ],
        out_specs=[
            pl.BlockSpec((1, 1, BT), lambda i: (i, 0, 0)),
            pl.BlockSpec((1, 1, BT), lambda i: (i, 0, 0)),
            pl.BlockSpec((1, 1, BT), lambda i: (i, 0, 0)),
            pl.BlockSpec((1, 1, BT), lambda i: (i, 0, 0)),
            pl.BlockSpec((1, 1, BT), lambda i: (i, 0, 0)),
            pl.BlockSpec((8, EPAD), lambda i: (0, 0)),
        ],
        out_shape=[
            jax.ShapeDtypeStruct((NSTEPS, 1, BT), jnp.int32),
            jax.ShapeDtypeStruct((NSTEPS, 1, BT), jnp.int32),
            jax.ShapeDtypeStruct((NSTEPS, 1, BT), jnp.int32),
            jax.ShapeDtypeStruct((NSTEPS, 1, BT), jnp.float32),
            jax.ShapeDtypeStruct((NSTEPS, 1, BT), jnp.float32),
            jax.ShapeDtypeStruct((8, EPAD), jnp.int32),
        ],
        scratch_shapes=[pltpu.VMEM((8, EPAD), jnp.float32)],
    )(x, wgp)


def _route_body(p1_hbm, e2_hbm, r2_hbm, cnt_hbm, x_hbm, p2_hbm, disp_hbm,
                p1_v, e2_v, r2_v, p2_v, cnt_v, xb0, xb1, semin, semsc):
    wid = lax.axis_index("s") * NC + lax.axis_index("c")
    rb = wid * 8
    tb = wid * TPW
    pltpu.sync_copy(p1_hbm.at[pl.ds(rb, 8)], p1_v)
    pltpu.sync_copy(e2_hbm.at[pl.ds(rb, 8)], e2_v)
    pltpu.sync_copy(r2_hbm.at[pl.ds(rb, 8)], r2_v)
    pltpu.sync_copy(cnt_hbm, cnt_v)
    xb = [xb0, xb1]
    din = [None] * 8
    din[0] = pltpu.async_copy(x_hbm.at[pl.ds(tb, CH)], xb0, semin)
    cntvec = cnt_v[...]
    for j in range(8):
        for k in range(2):
            sl = pl.ds(k * 16, 16)
            e2v = e2_v[j, sl]
            r2v = r2_v[j, sl]
            cnt_e = lax.gather(
                cntvec, e2v[:, None],
                lax.GatherDimensionNumbers(
                    offset_dims=(), collapsed_slice_dims=(0,),
                    start_index_map=(0,)),
                slice_sizes=(1,),
                mode=lax.GatherScatterMode.PROMISE_IN_BOUNDS)
            l2 = r2v - 1 + cnt_e
            p2_v[j, sl] = jnp.where(l2 < CAP, e2v * CAP + l2, SENT)
    pltpu.sync_copy(p2_v, p2_hbm.at[pl.ds(rb, 8)])
    d1 = [None] * 8
    d2 = [None] * 8
    for j in range(8):
        b = j % 2
        din[j].wait()
        d1[j] = pltpu.async_copy(xb[b], disp_hbm.at[p1_v.at[j]], semsc)
        d2[j] = pltpu.async_copy(xb[b], disp_hbm.at[p2_v.at[j]], semsc)
        if j + 1 < 8:
            if j - 1 >= 0:
                d1[j - 1].wait()
                d2[j - 1].wait()
            din[j + 1] = pltpu.async_copy(
                x_hbm.at[pl.ds(tb + (j + 1) * CH, CH)], xb[1 - b], semin)
    for j in (6, 7):
        d1[j].wait()
        d2[j].wait()


def _route_call(p1t, e2t, r2t, cnt16, x):
    mesh = plsc.VectorSubcoreMesh(
        core_axis_name="c", subcore_axis_name="s", num_cores=NC, num_subcores=NS)
    return pl.kernel(
        _route_body,
        out_type=[
            jax.ShapeDtypeStruct((N // CH, CH), jnp.int32),
            jax.ShapeDtypeStruct((SLOTS_PAD, D), jnp.float32),
        ],
        mesh=mesh,
        scratch_types=[
            pltpu.VMEM((8, CH), jnp.int32),
            pltpu.VMEM((8, CH), jnp.int32),
            pltpu.VMEM((8, CH), jnp.int32),
            pltpu.VMEM((8, CH), jnp.int32),
            pltpu.VMEM((16,), jnp.int32),
            pltpu.VMEM((CH, D), jnp.float32),
            pltpu.VMEM((CH, D), jnp.float32),
            pltpu.SemaphoreType.DMA,
            pltpu.SemaphoreType.DMA,
        ],
    )(p1t, e2t, r2t, cnt16, x)


def _ffn_body(de_ref, w1_ref, w2_ref, o_ref):
    e = pl.program_id(0)
    hstep = pl.program_id(1)

    @pl.when(e < E)
    def _():
        de = de_ref[...]
        w1 = w1_ref[0]
        w2 = w2_ref[0]
        hp = jnp.maximum(jnp.dot(de, w1, preferred_element_type=jnp.float32), 0.0)
        part = jnp.dot(hp, w2, preferred_element_type=jnp.float32)

        @pl.when(hstep == 0)
        def _():
            o_ref[...] = part

        @pl.when(hstep > 0)
        def _():
            o_ref[...] = o_ref[...] + part

    @pl.when((e == E) & (hstep == 0))
    def _():
        o_ref[...] = jnp.zeros_like(o_ref)


def _ffn_call(disp, W1, W2):
    return pl.pallas_call(
        _ffn_body,
        grid=(E + 1, H // BH),
        in_specs=[
            pl.BlockSpec((CAP, D), lambda e, h: (e, 0)),
            pl.BlockSpec((1, D, BH), lambda e, h: (jnp.minimum(e, E - 1), 0, h)),
            pl.BlockSpec((1, BH, D), lambda e, h: (jnp.minimum(e, E - 1), h, 0)),
        ],
        out_specs=pl.BlockSpec((CAP, D), lambda e, h: (e, 0)),
        out_shape=jax.ShapeDtypeStruct((SLOTS_PAD, D), jnp.float32),
    )(disp, W1, W2)


CHG = 16
NCHG = 16


def _vbcast(vec, t):
    idxv = jnp.full((16,), t, jnp.int32)
    return lax.gather(
        vec, idxv[:, None],
        lax.GatherDimensionNumbers(
            offset_dims=(), collapsed_slice_dims=(0,), start_index_map=(0,)),
        slice_sizes=(1,),
        mode=lax.GatherScatterMode.PROMISE_IN_BOUNDS)


def _combgather_body(seo_hbm, p1_hbm, p2_hbm, g1_hbm, g2_hbm, out_hbm,
                     p1_v, p2_v, g1_v, g2_v,
                     bA0, bA1, bB0, bB1, ob0, ob1, semg, semst):
    wid = lax.axis_index("s") * NC + lax.axis_index("c")
    rb = wid * NCHG
    tb = wid * TPW
    pltpu.sync_copy(p1_hbm.at[pl.ds(rb, NCHG)], p1_v)
    pltpu.sync_copy(p2_hbm.at[pl.ds(rb, NCHG)], p2_v)
    bA = [bA0, bA1]
    bB = [bB0, bB1]
    ob = [ob0, ob1]
    gA = [None] * NCHG
    gB = [None] * NCHG
    st = [None] * NCHG
    gA[0] = pltpu.async_copy(seo_hbm.at[p1_v.at[0]], bA0, semg)
    gB[0] = pltpu.async_copy(seo_hbm.at[p2_v.at[0]], bB0, semg)
    pltpu.sync_copy(g1_hbm.at[pl.ds(rb, NCHG)], g1_v)
    pltpu.sync_copy(g2_hbm.at[pl.ds(rb, NCHG)], g2_v)
    for j in range(NCHG):
        b = j % 2
        if j + 1 < NCHG:
            if j - 1 >= 0:
                st[j - 1].wait()
            gA[j + 1] = pltpu.async_copy(
                seo_hbm.at[p1_v.at[j + 1]], bA[1 - b], semg)
            gB[j + 1] = pltpu.async_copy(
                seo_hbm.at[p2_v.at[j + 1]], bB[1 - b], semg)
        gA[j].wait()
        gB[j].wait()
        ga = g1_v[j, pl.ds(0, CHG)]
        gb = g2_v[j, pl.ds(0, CHG)]
        bAb, bBb, obb = bA[b], bB[b], ob[b]

        def _tok(t, carry):
            gat = _vbcast(ga, t)
            gbt = _vbcast(gb, t)
            for q in range(D // 16):
                sl = pl.ds(q * 16, 16)
                obb[t, sl] = gat * bAb[t, sl] + gbt * bBb[t, sl]
            return carry

        lax.fori_loop(0, CHG, _tok, 0)
        st[j] = pltpu.async_copy(obb, out_hbm.at[pl.ds(tb + j * CHG, CHG)],
                                 semst)
    st[NCHG - 2].wait()
    st[NCHG - 1].wait()


def _combgather_call(seo, p1g, p2g, g1g, g2g):
    mesh = plsc.VectorSubcoreMesh(
        core_axis_name="c", subcore_axis_name="s", num_cores=NC, num_subcores=NS)
    return pl.kernel(
        _combgather_body,
        out_type=jax.ShapeDtypeStruct((N, D), jnp.float32),
        mesh=mesh,
        scratch_types=[
            pltpu.VMEM((NCHG, CHG), jnp.int32),
            pltpu.VMEM((NCHG, CHG), jnp.int32),
            pltpu.VMEM((NCHG, CHG), jnp.float32),
            pltpu.VMEM((NCHG, CHG), jnp.float32),
            pltpu.VMEM((CHG, D), jnp.float32),
            pltpu.VMEM((CHG, D), jnp.float32),
            pltpu.VMEM((CHG, D), jnp.float32),
            pltpu.VMEM((CHG, D), jnp.float32),
            pltpu.VMEM((CHG, D), jnp.float32),
            pltpu.VMEM((CHG, D), jnp.float32),
            pltpu.SemaphoreType.DMA,
            pltpu.SemaphoreType.DMA,
        ],
    )(seo, p1g, p2g, g1g, g2g)


def kernel(x, wg, W1, W2):
    wgp = jnp.pad(wg, ((0, 0), (0, EPAD - E)))
    p1r, e2r, r2r, g1r, g2r, cntr = _gating_call(x, wgp)
    p1t = p1r.reshape(N // CH, CH)
    e2t = e2r.reshape(N // CH, CH)
    r2t = r2r.reshape(N // CH, CH)
    cnt16 = cntr[0, :16]
    p2t, disp = _route_call(p1t, e2t, r2t, cnt16, x)
    seo = _ffn_call(disp, W1, W2)
    return _combgather_call(
        seo,
        p1t.reshape(N // CHG, CHG), p2t.reshape(N // CHG, CHG),
        g1r.reshape(N // CHG, CHG), g2r.reshape(N // CHG, CHG))


# speedup vs baseline: 1.5370x; 1.5370x over previous
import functools

import jax
import jax.numpy as jnp
from jax import lax
from jax.experimental import pallas as pl
from jax.experimental.pallas import tpu as pltpu
from jax.experimental.pallas import tpu_sc as plsc

N = 8192
D = 1024
E = 8
H = 4096
CAP = 2048
SLOTS = E * CAP
SENT = SLOTS
SLOTS_PAD = SLOTS + 16
BT = 256
NSTEPS = N // BT
BH = 512
EPAD = 128

D2 = D // 2
NC, NS = 2, 16
NW = NC * NS
TPW = N // NW
SPW = SLOTS // NW
CH = 32


def _f32_view(a):
    s = a.shape
    return lax.bitcast_convert_type(
        a.reshape(*s[:-1], s[-1] // 2, 2), jnp.float32)


def _bf16_view(a):
    s = a.shape
    return lax.bitcast_convert_type(a, jnp.bfloat16).reshape(*s[:-1], s[-1] * 2)


def _gating_body(x_ref, wg_ref, p1_ref, e2_ref, r2_ref, g1_ref, g2_ref,
                 cnt_ref, carry):
    step = pl.program_id(0)
    xb = x_ref[...]
    wgp = wg_ref[...]
    logits = jnp.dot(xb, wgp, preferred_element_type=jnp.float32)
    col = lax.broadcasted_iota(jnp.int32, (BT, EPAD), 1)
    validc = col < E
    neg = jnp.float32(-1e30)
    lg = jnp.where(validc, logits, neg)
    a1 = jnp.argmax(lg, axis=1).astype(jnp.int32)
    lg2 = jnp.where(col == a1[:, None], neg, lg)
    a2 = jnp.argmax(lg2, axis=1).astype(jnp.int32)
    mx = jnp.max(lg, axis=1, keepdims=True)
    ex = jnp.where(validc, jnp.exp(lg - mx), 0.0)
    den = jnp.sum(ex, axis=1)
    g1 = jnp.max(ex, axis=1) / den
    g2 = jnp.max(jnp.where(col == a1[:, None], 0.0, ex), axis=1) / den
    m1 = (col == a1[:, None]).astype(jnp.float32)
    m2 = (col == a2[:, None]).astype(jnp.float32)
    row_i = lax.broadcasted_iota(jnp.int32, (BT, BT), 0)
    col_i = lax.broadcasted_iota(jnp.int32, (BT, BT), 1)
    tri = (row_i >= col_i).astype(jnp.float32)

    @pl.when(step == 0)
    def _():
        carry[...] = jnp.zeros_like(carry)

    c1 = jnp.dot(tri, m1, preferred_element_type=jnp.float32) + carry[0:1, :]
    c2 = jnp.dot(tri, m2, preferred_element_type=jnp.float32) + carry[1:2, :]
    l1 = jnp.sum((c1 - 1.0) * m1, axis=1).astype(jnp.int32)
    r2i = jnp.sum(c2 * m2, axis=1).astype(jnp.int32)
    p1 = jnp.where(l1 < CAP, a1 * CAP + l1, SENT)
    carry[0:1, :] = c1[BT - 1:BT, :]
    carry[1:2, :] = c2[BT - 1:BT, :]
    p1_ref[...] = p1.reshape(1, 1, BT)
    e2_ref[...] = a2.reshape(1, 1, BT)
    r2_ref[...] = r2i.reshape(1, 1, BT)
    g1_ref[...] = g1.reshape(1, 1, BT)
    g2_ref[...] = g2.reshape(1, 1, BT)
    cnt_ref[...] = jnp.broadcast_to(c1[BT - 1:BT, :], (8, EPAD)).astype(jnp.int32)


def _gating_call(x, wgp):
    return pl.pallas_call(
        _gating_body,
        grid=(NSTEPS,),
        in_specs=[
            pl.BlockSpec((BT, D), lambda i: (i, 0)),
            pl.BlockSpec((D, EPAD), lambda i: (0, 0)),
        ],
        out_specs=[
            pl.BlockSpec((1, 1, BT), lambda i: (i, 0, 0)),
            pl.BlockSpec((1, 1, BT), lambda i: (i, 0, 0)),
            pl.BlockSpec((1, 1, BT), lambda i: (i, 0, 0)),
            pl.BlockSpec((1, 1, BT), lambda i: (i, 0, 0)),
            pl.BlockSpec((1, 1, BT), lambda i: (i, 0, 0)),
            pl.BlockSpec((8, EPAD), lambda i: (0, 0)),
        ],
        out_shape=[
            jax.ShapeDtypeStruct((NSTEPS, 1, BT), jnp.int32),
            jax.ShapeDtypeStruct((NSTEPS, 1, BT), jnp.int32),
            jax.ShapeDtypeStruct((NSTEPS, 1, BT), jnp.int32),
            jax.ShapeDtypeStruct((NSTEPS, 1, BT), jnp.float32),
            jax.ShapeDtypeStruct((NSTEPS, 1, BT), jnp.float32),
            jax.ShapeDtypeStruct((8, EPAD), jnp.int32),
        ],
        scratch_shapes=[pltpu.VMEM((8, EPAD), jnp.float32)],
    )(x, wgp)


def _route_body(p1_hbm, e2_hbm, r2_hbm, cnt_hbm, x_hbm, p2_hbm, disp_hbm,
                p1_v, e2_v, r2_v, p2_v, cnt_v, xb0, xb1, semin, semsc):
    wid = lax.axis_index("s") * NC + lax.axis_index("c")
    rb = wid * 8
    tb = wid * TPW
    pltpu.sync_copy(p1_hbm.at[pl.ds(rb, 8)], p1_v)
    pltpu.sync_copy(e2_hbm.at[pl.ds(rb, 8)], e2_v)
    pltpu.sync_copy(r2_hbm.at[pl.ds(rb, 8)], r2_v)
    pltpu.sync_copy(cnt_hbm, cnt_v)
    xb = [xb0, xb1]
    din = [None] * 8
    din[0] = pltpu.async_copy(x_hbm.at[pl.ds(tb, CH)], xb0, semin)
    cntvec = cnt_v[...]
    for j in range(8):
        for k in range(2):
            sl = pl.ds(k * 16, 16)
            e2v = e2_v[j, sl]
            r2v = r2_v[j, sl]
            cnt_e = lax.gather(
                cntvec, e2v[:, None],
                lax.GatherDimensionNumbers(
                    offset_dims=(), collapsed_slice_dims=(0,),
                    start_index_map=(0,)),
                slice_sizes=(1,),
                mode=lax.GatherScatterMode.PROMISE_IN_BOUNDS)
            l2 = r2v - 1 + cnt_e
            p2_v[j, sl] = jnp.where(l2 < CAP, e2v * CAP + l2, SENT)
    pltpu.sync_copy(p2_v, p2_hbm.at[pl.ds(rb, 8)])
    d1 = [None] * 8
    d2 = [None] * 8
    for j in range(8):
        b = j % 2
        din[j].wait()
        d1[j] = pltpu.async_copy(xb[b], disp_hbm.at[p1_v.at[j]], semsc)
        d2[j] = pltpu.async_copy(xb[b], disp_hbm.at[p2_v.at[j]], semsc)
        if j + 1 < 8:
            if j - 1 >= 0:
                d1[j - 1].wait()
                d2[j - 1].wait()
            din[j + 1] = pltpu.async_copy(
                x_hbm.at[pl.ds(tb + (j + 1) * CH, CH)], xb[1 - b], semin)
    for j in (6, 7):
        d1[j].wait()
        d2[j].wait()


def _route_call(p1t, e2t, r2t, cnt16, x):
    mesh = plsc.VectorSubcoreMesh(
        core_axis_name="c", subcore_axis_name="s", num_cores=NC, num_subcores=NS)
    return pl.kernel(
        _route_body,
        out_type=[
            jax.ShapeDtypeStruct((N // CH, CH), jnp.int32),
            jax.ShapeDtypeStruct((SLOTS_PAD, D), jnp.float32),
        ],
        mesh=mesh,
        scratch_types=[
            pltpu.VMEM((8, CH), jnp.int32),
            pltpu.VMEM((8, CH), jnp.int32),
            pltpu.VMEM((8, CH), jnp.int32),
            pltpu.VMEM((8, CH), jnp.int32),
            pltpu.VMEM((16,), jnp.int32),
            pltpu.VMEM((CH, D), jnp.float32),
            pltpu.VMEM((CH, D), jnp.float32),
            pltpu.SemaphoreType.DMA,
            pltpu.SemaphoreType.DMA,
        ],
    )(p1t, e2t, r2t, cnt16, x)


def _ffn_body(de_ref, w1_ref, w2_ref, o_ref):
    e = pl.program_id(0)
    hstep = pl.program_id(1)

    @pl.when(e < E)
    def _():
        de = de_ref[...]
        w1 = w1_ref[0]
        w2 = w2_ref[0]
        hp = jnp.maximum(jnp.dot(de, w1, preferred_element_type=jnp.float32), 0.0)
        part = jnp.dot(hp, w2, preferred_element_type=jnp.float32)

        @pl.when(hstep == 0)
        def _():
            o_ref[...] = part

        @pl.when(hstep > 0)
        def _():
            o_ref[...] = o_ref[...] + part

    @pl.when((e == E) & (hstep == 0))
    def _():
        o_ref[...] = jnp.zeros_like(o_ref)


def _ffn_call(disp, W1, W2):
    return pl.pallas_call(
        _ffn_body,
        grid=(E + 1, H // BH),
        in_specs=[
            pl.BlockSpec((CAP, D), lambda e, h: (e, 0)),
            pl.BlockSpec((1, D, BH), lambda e, h: (jnp.minimum(e, E - 1), 0, h)),
            pl.BlockSpec((1, BH, D), lambda e, h: (jnp.minimum(e, E - 1), h, 0)),
        ],
        out_specs=pl.BlockSpec((CAP, D), lambda e, h: (e, 0)),
        out_shape=jax.ShapeDtypeStruct((SLOTS_PAD, D), jnp.float32),
    )(disp, W1, W2)


CHG = 16
NCHG = 16


def _vbcast(vec, t):
    idxv = jnp.full((16,), t, jnp.int32)
    return lax.gather(
        vec, idxv[:, None],
        lax.GatherDimensionNumbers(
            offset_dims=(), collapsed_slice_dims=(0,), start_index_map=(0,)),
        slice_sizes=(1,),
        mode=lax.GatherScatterMode.PROMISE_IN_BOUNDS)


def _combgather_body(seo_hbm, p1_hbm, p2_hbm, g1_hbm, g2_hbm, out_hbm,
                     p1_v, p2_v, g1_v, g2_v,
                     bA0, bA1, bB0, bB1, ob0, ob1, semg, semst):
    wid = lax.axis_index("s") * NC + lax.axis_index("c")
    rb = wid * NCHG
    tb = wid * TPW
    pltpu.sync_copy(p1_hbm.at[pl.ds(rb, NCHG)], p1_v)
    pltpu.sync_copy(p2_hbm.at[pl.ds(rb, NCHG)], p2_v)
    bA = [bA0, bA1]
    bB = [bB0, bB1]
    ob = [ob0, ob1]
    gA = [None] * NCHG
    gB = [None] * NCHG
    st = [None] * NCHG
    gA[0] = pltpu.async_copy(seo_hbm.at[p1_v.at[0]], bA0, semg)
    gB[0] = pltpu.async_copy(seo_hbm.at[p2_v.at[0]], bB0, semg)
    pltpu.sync_copy(g1_hbm.at[pl.ds(rb, NCHG)], g1_v)
    pltpu.sync_copy(g2_hbm.at[pl.ds(rb, NCHG)], g2_v)
    for j in range(NCHG):
        b = j % 2
        if j + 1 < NCHG:
            if j - 1 >= 0:
                st[j - 1].wait()
            gA[j + 1] = pltpu.async_copy(
                seo_hbm.at[p1_v.at[j + 1]], bA[1 - b], semg)
            gB[j + 1] = pltpu.async_copy(
                seo_hbm.at[p2_v.at[j + 1]], bB[1 - b], semg)
        gA[j].wait()
        gB[j].wait()
        ga = g1_v[j, pl.ds(0, CHG)]
        gb = g2_v[j, pl.ds(0, CHG)]
        bAb, bBb, obb = bA[b], bB[b], ob[b]

        def _tok(t, carry):
            gat = _vbcast(ga, t)
            gbt = _vbcast(gb, t)
            for q in range(D // 16):
                sl = pl.ds(q * 16, 16)
                obb[t, sl] = gat * bAb[t, sl] + gbt * bBb[t, sl]
            return carry

        lax.fori_loop(0, CHG, _tok, 0)
        st[j] = pltpu.async_copy(obb, out_hbm.at[pl.ds(tb + j * CHG, CHG)],
                                 semst)
    st[NCHG - 2].wait()
    st[NCHG - 1].wait()


def _combgather_call(seo, p1g, p2g, g1g, g2g):
    mesh = plsc.VectorSubcoreMesh(
        core_axis_name="c", subcore_axis_name="s", num_cores=NC, num_subcores=NS)
    return pl.kernel(
        _combgather_body,
        out_type=jax.ShapeDtypeStruct((N, D), jnp.float32),
        mesh=mesh,
        scratch_types=[
            pltpu.VMEM((NCHG, CHG), jnp.int32),
            pltpu.VMEM((NCHG, CHG), jnp.int32),
            pltpu.VMEM((NCHG, CHG), jnp.float32),
            pltpu.VMEM((NCHG, CHG), jnp.float32),
            pltpu.VMEM((CHG, D), jnp.float32),
            pltpu.VMEM((CHG, D), jnp.float32),
            pltpu.VMEM((CHG, D), jnp.float32),
            pltpu.VMEM((CHG, D), jnp.float32),
            pltpu.VMEM((CHG, D), jnp.float32),
            pltpu.VMEM((CHG, D), jnp.float32),
            pltpu.SemaphoreType.DMA,
            pltpu.SemaphoreType.DMA,
        ],
    )(seo, p1g, p2g, g1g, g2g)


def kernel(x, wg, W1, W2):
    wgp = jnp.pad(wg, ((0, 0), (0, EPAD - E)))
    p1r, e2r, r2r, g1r, g2r, cntr = _gating_call(x, wgp)
    p1t = p1r.reshape(N // CH, CH)
    e2t = e2r.reshape(N // CH, CH)
    r2t = r2r.reshape(N // CH, CH)
    cnt16 = cntr[0, :16]
    p2t, disp = _route_call(p1t, e2t, r2t, cnt16, x)
    seo = _ffn_call(disp, W1, W2)
    return _combgather_call(
        seo,
        p1t.reshape(N // CHG, CHG), p2t.reshape(N // CHG, CHG),
        g1r.reshape(N // CHG, CHG), g2r.reshape(N // CHG, CHG))
